# baseline (speedup 1.0000x reference)
import functools

import jax
import jax.numpy as jnp
from jax import lax
from jax.experimental import pallas as pl
from jax.experimental.pallas import tpu as pltpu
from jax.experimental.pallas import tpu_sc as plsc

HMAP = 128
WMAP = 128
N = HMAP * WMAP
K = 1024
B_ROWS = 16
A_COLS = 1024

WORD_MIN_SCORE = 0.5
CHAR_MIN_SCORE = 0.25
WORD_NMS_IOU = 0.5
CHAR_NMS_IOU = 0.3
NUM_CHAR_CLASS = 68
STRIDE = 4.0

NC, NS, L = 2, 16, 16
NW = NC * NS
BPW = K // NW
N_WCH = 5
N_CCH = 4 + NUM_CHAR_CLASS
N_CH = N_WCH + N_CCH


def _roll(x, s, axis):
    n = x.shape[axis]
    s = s % n
    if s == 0:
        return x
    lo = [slice(None)] * x.ndim
    hi = [slice(None)] * x.ndim
    lo[axis] = slice(s, n)
    hi[axis] = slice(0, s)
    return jnp.concatenate([x[tuple(lo)], x[tuple(hi)]], axis=axis)


def _bitonic_topk(keys, idx, b_io, a_io):
    k = 2
    while k <= N:
        j = k // 2
        while j >= 1:
            if j < A_COLS:
                axis, sh = 2, j
                has_bit = (a_io & j) != 0
            else:
                axis, sh = 1, j >> 10
                has_bit = (b_io & (j >> 10)) != 0
            if k < A_COLS:
                desc = (a_io & k) == 0
            else:
                desc = (b_io & (k >> 10)) == 0
            pk_up = _roll(keys, sh, axis)
            pk_dn = _roll(keys, -sh, axis)
            pi_up = _roll(idx, sh, axis)
            pi_dn = _roll(idx, -sh, axis)
            pk = jnp.where(has_bit, pk_dn, pk_up)
            pi = jnp.where(has_bit, pi_dn, pi_up)
            mine_larger = (keys > pk) | ((keys == pk) & (idx < pi))
            want_larger = desc != has_bit
            take_mine = mine_larger == want_larger
            keys = jnp.where(take_mine, keys, pk)
            idx = jnp.where(take_mine, idx, pi)
            j //= 2
        k *= 2
    return keys, idx


def _sort_body(wf_ref, cf_ref, topv_ref, topi_ref):
    wf = wf_ref[...]
    cf = cf_ref[...]
    wscore = jnp.where(wf > WORD_MIN_SCORE, wf, 0.0)
    cscore = jnp.where((wf > WORD_MIN_SCORE) & (cf > CHAR_MIN_SCORE), cf, 0.0)
    keys = jnp.stack([wscore, cscore], axis=0)
    b_io = lax.broadcasted_iota(jnp.int32, (2, B_ROWS, A_COLS), 1)
    a_io = lax.broadcasted_iota(jnp.int32, (2, B_ROWS, A_COLS), 2)
    idx = b_io * A_COLS + a_io
    keys, idx = _bitonic_topk(keys, idx, b_io, a_io)
    topv_ref[...] = keys[:, 0, :]
    topi_ref[...] = idx[:, 0, :]


def _sc_gather_body(tab_ref, widx_ref, cidx_ref, out_ref,
                    widx_v, cidx_v, idx_all, rows_v, sem):
    wid = lax.axis_index("s") * NC + lax.axis_index("c")
    base = wid * BPW
    pltpu.sync_copy(widx_ref.at[pl.ds(base, BPW)], widx_v)
    pltpu.sync_copy(cidx_ref.at[pl.ds(base, BPW)], cidx_v)
    for ch in range(N_CH):
        src = widx_v if ch < N_WCH else cidx_v
        off = jnp.int32(ch * N)
        for g in range(BPW // L):
            idx_all[ch, pl.ds(g * L, L)] = src[pl.ds(g * L, L)] + off
    copies = [
        pltpu.async_copy(tab_ref.at[idx_all.at[ch]], rows_v.at[ch], sem)
        for ch in range(N_CH)
    ]
    for cp in copies:
        cp.wait()
    pltpu.sync_copy(rows_v, out_ref.at[wid])


_sc_gather = pl.kernel(
    _sc_gather_body,
    out_type=jax.ShapeDtypeStruct((NW, N_CH, BPW), jnp.float32),
    mesh=plsc.VectorSubcoreMesh(core_axis_name="c", subcore_axis_name="s",
                                num_cores=NC, num_subcores=NS),
    scratch_types=[
        pltpu.VMEM((BPW,), jnp.int32),
        pltpu.VMEM((BPW,), jnp.int32),
        pltpu.VMEM((N_CH, BPW), jnp.int32),
        pltpu.VMEM((N_CH, BPW), jnp.float32),
        pltpu.SemaphoreType.DMA,
    ],
)


def _transpose(x):
    return x.T


def _nms_and_pack(qx, qy, topv, iou_thresh, wm1, hm1):
    bx1 = jnp.minimum(jnp.minimum(qx[0], qx[1]), jnp.minimum(qx[2], qx[3]))
    bx2 = jnp.maximum(jnp.maximum(qx[0], qx[1]), jnp.maximum(qx[2], qx[3]))
    by1 = jnp.minimum(jnp.minimum(qy[0], qy[1]), jnp.minimum(qy[2], qy[3]))
    by2 = jnp.maximum(jnp.maximum(qy[0], qy[1]), jnp.maximum(qy[2], qy[3]))
    area = (bx2 - bx1) * (by2 - by1)
    validf = topv > 0.0

    cols = jnp.concatenate([bx1, by1, bx2, by2, area, topv], axis=0)
    cols_t = _transpose(cols)
    bx1c = cols_t[:, 0:1]
    by1c = cols_t[:, 1:2]
    bx2c = cols_t[:, 2:3]
    by2c = cols_t[:, 3:4]
    areac = cols_t[:, 4:5]
    validc = cols_t[:, 5:6] > 0.0

    ix1 = jnp.maximum(bx1c, bx1)
    iy1 = jnp.maximum(by1c, by1)
    ix2 = jnp.minimum(bx2c, bx2)
    iy2 = jnp.minimum(by2c, by2)
    inter = jnp.maximum(ix2 - ix1, 0.0) * jnp.maximum(iy2 - iy1, 0.0)
    iou = inter / (areac + area - inter + 1e-6)

    sub_io = lax.broadcasted_iota(jnp.int32, (K, K), 0)
    lane_io = lax.broadcasted_iota(jnp.int32, (K, K), 1)
    sup = (iou > iou_thresh) & (sub_io < lane_io) & validc
    supf = jnp.where(sup, 1.0, 0.0)
    suppressed = jnp.max(supf, axis=0, keepdims=True)
    keepf = jnp.where(validf & (suppressed < 0.5), 1.0, 0.0)

    rows = []
    for i in range(4):
        rows.append(jnp.clip(jnp.round(qx[i]), 0.0, wm1))
        rows.append(jnp.clip(jnp.round(qy[i]), 0.0, hm1))
    rows.append(topv)
    out9 = jnp.concatenate(rows, axis=0) * keepf
    return out9, keepf


def _post_body(topv_ref, topi_ref, feats_ref, params_ref,
               cb_ref, cs_ref, wb_ref):
    topv = topv_ref[...]
    topi = topi_ref[...]
    feats = feats_ref[...]
    params = params_ref[...]
    sw = params[:, 0:1]
    sh = params[:, 1:2]
    wm1 = params[:, 2:3]
    hm1 = params[:, 3:4]
    sw4 = sw * STRIDE
    sh4 = sh * STRIDE

    wtopv = topv[0:1, :]
    wtopi = topi[0:1, :]
    t, b, l, r = feats[0:1], feats[1:2], feats[2:3], feats[3:4]
    orient = feats[4:5]
    xs = (wtopi % WMAP).astype(jnp.float32)
    ys = (wtopi // WMAP).astype(jnp.float32)
    x1 = sw4 * (xs - l)
    y1 = sh4 * (ys - t)
    x2 = sw4 * (xs + r)
    y2 = sh4 * (ys + b)
    cx = sw4 * xs
    cy = sh4 * ys
    cosv = jnp.cos(orient)
    sinv = jnp.sin(orient)
    dx1 = x1 - cx
    dx2 = x2 - cx
    dy1 = y1 - cy
    dy2 = y2 - cy
    qx = [cx + cosv * dx1 - sinv * dy1,
          cx + cosv * dx2 - sinv * dy1,
          cx + cosv * dx2 - sinv * dy2,
          cx + cosv * dx1 - sinv * dy2]
    qy = [cy + sinv * dx1 + cosv * dy1,
          cy + sinv * dx2 + cosv * dy1,
          cy + sinv * dx2 + cosv * dy2,
          cy + sinv * dx1 + cosv * dy2]
    wout9, _ = _nms_and_pack(qx, qy, wtopv, WORD_NMS_IOU, wm1, hm1)
    wb_ref[...] = _transpose(wout9)

    ctopv = topv[1:2, :]
    ctopi = topi[1:2, :]
    ct, cb, cl, cr = feats[5:6], feats[6:7], feats[7:8], feats[8:9]
    cls = feats[9:9 + NUM_CHAR_CLASS]
    cxs = (ctopi % WMAP).astype(jnp.float32)
    cys = (ctopi // WMAP).astype(jnp.float32)
    cx1 = sw4 * (cxs - cl)
    cy1 = sh4 * (cys - ct)
    cx2 = sw4 * (cxs + cr)
    cy2 = sh4 * (cys + cb)
    cqx = [cx1, cx2, cx2, cx1]
    cqy = [cy1, cy1, cy2, cy2]
    cout9, ckeep = _nms_and_pack(cqx, cqy, ctopv, CHAR_NMS_IOU, wm1, hm1)
    cb_ref[...] = _transpose(cout9)
    cs_ref[...] = _transpose(cls * ckeep)


@functools.partial(jax.jit, static_argnums=())
def kernel(pred_word_fg, pred_word_tblr, pred_word_orient, pred_char_fg,
           pred_char_tblr, pred_char_cls, im_scale_w, im_scale_h,
           original_im_w, original_im_h):
    wf = pred_word_fg.reshape(B_ROWS, A_COLS)
    cf = pred_char_fg.reshape(B_ROWS, A_COLS)
    tab = jnp.concatenate([
        pred_word_tblr.reshape(4, N),
        pred_word_orient.reshape(1, N),
        pred_char_tblr.reshape(4, N),
        pred_char_cls.reshape(NUM_CHAR_CLASS, N),
    ], axis=0).reshape(N_CH * N)
    wm1 = jnp.asarray(original_im_w, jnp.float32) - 1.0
    hm1 = jnp.asarray(original_im_h, jnp.float32) - 1.0
    params = jnp.stack([
        jnp.asarray(im_scale_w, jnp.float32),
        jnp.asarray(im_scale_h, jnp.float32),
        wm1, hm1,
    ]).reshape(1, 4)

    topv2, topi2 = pl.pallas_call(
        _sort_body,
        out_shape=(
            jax.ShapeDtypeStruct((2, K), jnp.float32),
            jax.ShapeDtypeStruct((2, K), jnp.int32),
        ),
    )(wf, cf)

    feats3 = _sc_gather(tab, topi2[0], topi2[1])
    feats = feats3.transpose(1, 0, 2).reshape(N_CH, K)

    out_shapes = (
        jax.ShapeDtypeStruct((K, 9), jnp.float32),
        jax.ShapeDtypeStruct((K, NUM_CHAR_CLASS), jnp.float32),
        jax.ShapeDtypeStruct((K, 9), jnp.float32),
    )
    char_bboxes, char_scores, word_bboxes = pl.pallas_call(
        _post_body,
        out_shape=out_shapes,
    )(topv2, topi2, feats, params)
    return (char_bboxes, char_scores, word_bboxes)


# speedup vs baseline: 1.0875x; 1.0875x over previous
import functools

import jax
import jax.numpy as jnp
from jax import lax
from jax.experimental import pallas as pl
from jax.experimental.pallas import tpu as pltpu
from jax.experimental.pallas import tpu_sc as plsc

HMAP = 128
WMAP = 128
N = HMAP * WMAP
K = 1024
B_ROWS = 16
A_COLS = 1024

WORD_MIN_SCORE = 0.5
CHAR_MIN_SCORE = 0.25
WORD_NMS_IOU = 0.5
CHAR_NMS_IOU = 0.3
NUM_CHAR_CLASS = 68
STRIDE = 4.0

NC, NS, L = 2, 16, 16
NW = NC * NS
BPW = K // NW


def _roll(x, s, axis):
    n = x.shape[axis]
    s = s % n
    if s == 0:
        return x
    lo = [slice(None)] * x.ndim
    hi = [slice(None)] * x.ndim
    lo[axis] = slice(s, n)
    hi[axis] = slice(0, s)
    return jnp.concatenate([x[tuple(lo)], x[tuple(hi)]], axis=axis)


def _bitonic_topk(keys):
    b_io = lax.broadcasted_iota(jnp.int32, (B_ROWS, A_COLS), 0)
    a_io = lax.broadcasted_iota(jnp.int32, (B_ROWS, A_COLS), 1)
    idx = b_io * A_COLS + a_io
    k = 2
    while k <= N:
        j = k // 2
        while j >= 1:
            if j < A_COLS:
                axis, sh = 1, j
                has_bit = (a_io & j) != 0
            else:
                axis, sh = 0, j >> 10
                has_bit = (b_io & (j >> 10)) != 0
            if k < A_COLS:
                desc = (a_io & k) == 0
            else:
                desc = (b_io & (k >> 10)) == 0
            pk = jnp.where(has_bit, _roll(keys, -sh, axis), _roll(keys, sh, axis))
            pi = jnp.where(has_bit, _roll(idx, -sh, axis), _roll(idx, sh, axis))
            mine_larger = (keys > pk) | ((keys == pk) & (idx < pi))
            take_mine = mine_larger == (desc != has_bit)
            keys = jnp.where(take_mine, keys, pk)
            idx = jnp.where(take_mine, idx, pi)
            j //= 2
        k *= 2
    return keys, idx


def _char_sort_body(wf_ref, cf_ref, topv_ref, topi_ref):
    wf = wf_ref[...]
    cf = cf_ref[...]
    cscore = jnp.where((wf > WORD_MIN_SCORE) & (cf > CHAR_MIN_SCORE), cf, 0.0)
    keys, idx = _bitonic_topk(cscore)
    topv_ref[...] = keys[0:1, :]
    topi_ref[...] = idx[0:1, :]


def _word_sort_body(wf_ref, topv_ref, topi_ref):
    wf = wf_ref[...]
    wscore = jnp.where(wf > WORD_MIN_SCORE, wf, 0.0)
    keys, idx = _bitonic_topk(wscore)
    topv_ref[...] = keys[0:1, :]
    topi_ref[...] = idx[0:1, :]


def _sc_gather_body(cls_ref, cidx_ref, out_ref, cidx_v, idx_all, rows_v, sem):
    wid = lax.axis_index("s") * NC + lax.axis_index("c")
    base = wid * BPW
    pltpu.sync_copy(cidx_ref.at[pl.ds(base, BPW)], cidx_v)
    for ch in range(NUM_CHAR_CLASS):
        off = jnp.int32(ch * N)
        for g in range(BPW // L):
            idx_all[ch, pl.ds(g * L, L)] = cidx_v[pl.ds(g * L, L)] + off
    copies = [
        pltpu.async_copy(cls_ref.at[idx_all.at[ch]], rows_v.at[ch], sem)
        for ch in range(NUM_CHAR_CLASS)
    ]
    for cp in copies:
        cp.wait()
    pltpu.sync_copy(rows_v, out_ref.at[wid])


_sc_gather = pl.kernel(
    _sc_gather_body,
    out_type=jax.ShapeDtypeStruct((NW, NUM_CHAR_CLASS, BPW), jnp.float32),
    mesh=plsc.VectorSubcoreMesh(core_axis_name="c", subcore_axis_name="s",
                                num_cores=NC, num_subcores=NS),
    scratch_types=[
        pltpu.VMEM((BPW,), jnp.int32),
        pltpu.VMEM((NUM_CHAR_CLASS, BPW), jnp.int32),
        pltpu.VMEM((NUM_CHAR_CLASS, BPW), jnp.float32),
        pltpu.SemaphoreType.DMA,
    ],
)


def _transpose(x):
    return x.T


def _gather_feats(stack, topi, n_ch):
    r = topi // WMAP
    c = topi % WMAP
    sub_io = lax.broadcasted_iota(jnp.int32, (HMAP, K), 0)
    row_sel = jnp.where(sub_io == jnp.broadcast_to(r, (HMAP, K)), 1.0, 0.0)
    col_sel = jnp.where(sub_io == jnp.broadcast_to(c, (HMAP, K)), 1.0, 0.0)
    h = jnp.dot(stack, col_sel, preferred_element_type=jnp.float32,
                precision=lax.Precision.HIGHEST)
    feats = []
    for ci in range(n_ch):
        hm = h[ci * HMAP:(ci + 1) * HMAP, :] * row_sel
        feats.append(jnp.sum(hm, axis=0, keepdims=True))
    return feats


def _nms_and_pack(qx, qy, topv, iou_thresh, wm1, hm1):
    bx1 = jnp.minimum(jnp.minimum(qx[0], qx[1]), jnp.minimum(qx[2], qx[3]))
    bx2 = jnp.maximum(jnp.maximum(qx[0], qx[1]), jnp.maximum(qx[2], qx[3]))
    by1 = jnp.minimum(jnp.minimum(qy[0], qy[1]), jnp.minimum(qy[2], qy[3]))
    by2 = jnp.maximum(jnp.maximum(qy[0], qy[1]), jnp.maximum(qy[2], qy[3]))
    area = (bx2 - bx1) * (by2 - by1)
    validf = topv > 0.0

    cols = jnp.concatenate([bx1, by1, bx2, by2, area, topv], axis=0)
    cols_t = _transpose(cols)
    bx1c = cols_t[:, 0:1]
    by1c = cols_t[:, 1:2]
    bx2c = cols_t[:, 2:3]
    by2c = cols_t[:, 3:4]
    areac = cols_t[:, 4:5]
    validc = cols_t[:, 5:6] > 0.0

    ix1 = jnp.maximum(bx1c, bx1)
    iy1 = jnp.maximum(by1c, by1)
    ix2 = jnp.minimum(bx2c, bx2)
    iy2 = jnp.minimum(by2c, by2)
    inter = jnp.maximum(ix2 - ix1, 0.0) * jnp.maximum(iy2 - iy1, 0.0)
    iou = inter / (areac + area - inter + 1e-6)

    sub_io = lax.broadcasted_iota(jnp.int32, (K, K), 0)
    lane_io = lax.broadcasted_iota(jnp.int32, (K, K), 1)
    sup = (iou > iou_thresh) & (sub_io < lane_io) & validc
    supf = jnp.where(sup, 1.0, 0.0)
    suppressed = jnp.max(supf, axis=0, keepdims=True)
    keepf = jnp.where(validf & (suppressed < 0.5), 1.0, 0.0)

    rows = []
    for i in range(4):
        rows.append(jnp.clip(jnp.round(qx[i]), 0.0, wm1))
        rows.append(jnp.clip(jnp.round(qy[i]), 0.0, hm1))
    rows.append(topv)
    out9 = jnp.concatenate(rows, axis=0) * keepf
    return out9, keepf


def _box_body(wtopv_ref, wtopi_ref, ctopv_ref, ctopi_ref,
              wstack_ref, cstack_ref, params_ref,
              cb_ref, wb_ref, keep_ref):
    params = params_ref[...]
    sw4 = params[:, 0:1] * STRIDE
    sh4 = params[:, 1:2] * STRIDE
    wm1 = params[:, 2:3]
    hm1 = params[:, 3:4]

    wtopv = wtopv_ref[...]
    wtopi = wtopi_ref[...]
    t, b, l, r, orient = _gather_feats(wstack_ref[...], wtopi, 5)
    xs = (wtopi % WMAP).astype(jnp.float32)
    ys = (wtopi // WMAP).astype(jnp.float32)
    x1 = sw4 * (xs - l)
    y1 = sh4 * (ys - t)
    x2 = sw4 * (xs + r)
    y2 = sh4 * (ys + b)
    cx = sw4 * xs
    cy = sh4 * ys
    cosv = jnp.cos(orient)
    sinv = jnp.sin(orient)
    dx1 = x1 - cx
    dx2 = x2 - cx
    dy1 = y1 - cy
    dy2 = y2 - cy
    qx = [cx + cosv * dx1 - sinv * dy1,
          cx + cosv * dx2 - sinv * dy1,
          cx + cosv * dx2 - sinv * dy2,
          cx + cosv * dx1 - sinv * dy2]
    qy = [cy + sinv * dx1 + cosv * dy1,
          cy + sinv * dx2 + cosv * dy1,
          cy + sinv * dx2 + cosv * dy2,
          cy + sinv * dx1 + cosv * dy2]
    wout9, _ = _nms_and_pack(qx, qy, wtopv, WORD_NMS_IOU, wm1, hm1)
    wb_ref[...] = _transpose(wout9)

    ctopv = ctopv_ref[...]
    ctopi = ctopi_ref[...]
    ct, cb, cl, cr = _gather_feats(cstack_ref[...], ctopi, 4)
    cxs = (ctopi % WMAP).astype(jnp.float32)
    cys = (ctopi // WMAP).astype(jnp.float32)
    cx1 = sw4 * (cxs - cl)
    cy1 = sh4 * (cys - ct)
    cx2 = sw4 * (cxs + cr)
    cy2 = sh4 * (cys + cb)
    cqx = [cx1, cx2, cx2, cx1]
    cqy = [cy1, cy1, cy2, cy2]
    cout9, ckeep = _nms_and_pack(cqx, cqy, ctopv, CHAR_NMS_IOU, wm1, hm1)
    cb_ref[...] = _transpose(cout9)
    keep_ref[...] = ckeep


def _finish_body(cls_ref, keep_ref, cs_ref):
    cs_ref[...] = _transpose(cls_ref[...] * keep_ref[...])


@functools.partial(jax.jit, static_argnums=())
def kernel(pred_word_fg, pred_word_tblr, pred_word_orient, pred_char_fg,
           pred_char_tblr, pred_char_cls, im_scale_w, im_scale_h,
           original_im_w, original_im_h):
    wf = pred_word_fg.reshape(B_ROWS, A_COLS)
    cf = pred_char_fg.reshape(B_ROWS, A_COLS)
    cls_flat = pred_char_cls.reshape(NUM_CHAR_CLASS * N)
    wstack = jnp.concatenate(
        [pred_word_tblr, pred_word_orient[None]], axis=0
    ).reshape(5 * HMAP, WMAP)
    cstack = pred_char_tblr.reshape(4 * HMAP, WMAP)
    wm1 = jnp.asarray(original_im_w, jnp.float32) - 1.0
    hm1 = jnp.asarray(original_im_h, jnp.float32) - 1.0
    params = jnp.stack([
        jnp.asarray(im_scale_w, jnp.float32),
        jnp.asarray(im_scale_h, jnp.float32),
        wm1, hm1,
    ]).reshape(1, 4)

    ctopv, ctopi = pl.pallas_call(
        _char_sort_body,
        out_shape=(
            jax.ShapeDtypeStruct((1, K), jnp.float32),
            jax.ShapeDtypeStruct((1, K), jnp.int32),
        ),
    )(wf, cf)

    cls3 = _sc_gather(cls_flat, ctopi.reshape(K))

    wtopv, wtopi = pl.pallas_call(
        _word_sort_body,
        out_shape=(
            jax.ShapeDtypeStruct((1, K), jnp.float32),
            jax.ShapeDtypeStruct((1, K), jnp.int32),
        ),
    )(wf)

    char_bboxes, word_bboxes, ckeep = pl.pallas_call(
        _box_body,
        out_shape=(
            jax.ShapeDtypeStruct((K, 9), jnp.float32),
            jax.ShapeDtypeStruct((K, 9), jnp.float32),
            jax.ShapeDtypeStruct((1, K), jnp.float32),
        ),
    )(wtopv, wtopi, ctopv, ctopi, wstack, cstack, params)

    cls2d = cls3.transpose(1, 0, 2).reshape(NUM_CHAR_CLASS, K)
    char_scores = pl.pallas_call(
        _finish_body,
        out_shape=jax.ShapeDtypeStruct((K, NUM_CHAR_CLASS), jnp.float32),
    )(cls2d, ckeep)
    return (char_bboxes, char_scores, word_bboxes)


# speedup vs baseline: 1.1339x; 1.0426x over previous
import functools

import jax
import jax.numpy as jnp
from jax import lax
from jax.experimental import pallas as pl
from jax.experimental.pallas import tpu as pltpu
from jax.experimental.pallas import tpu_sc as plsc

HMAP = 128
WMAP = 128
N = HMAP * WMAP
K = 1024
B_ROWS = 16
A_COLS = 1024

WORD_MIN_SCORE = 0.5
CHAR_MIN_SCORE = 0.25
WORD_NMS_IOU = 0.5
CHAR_NMS_IOU = 0.3
NUM_CHAR_CLASS = 68
STRIDE = 4.0

NC, NS, L = 2, 16, 16
NW = NC * NS
BPW = K // NW


def _roll(x, s, axis):
    n = x.shape[axis]
    s = s % n
    if s == 0:
        return x
    lo = [slice(None)] * x.ndim
    hi = [slice(None)] * x.ndim
    lo[axis] = slice(s, n)
    hi[axis] = slice(0, s)
    return jnp.concatenate([x[tuple(lo)], x[tuple(hi)]], axis=axis)


def _cmpx(keys, idx, j, a_io, desc):
    has_bit = (a_io & j) != 0
    pk = jnp.where(has_bit, _roll(keys, -j, 1), _roll(keys, j, 1))
    pi = jnp.where(has_bit, _roll(idx, -j, 1), _roll(idx, j, 1))
    mine_larger = (keys > pk) | ((keys == pk) & (idx < pi))
    take_mine = mine_larger == (desc != has_bit)
    return jnp.where(take_mine, keys, pk), jnp.where(take_mine, idx, pi)


def _bitonic_topk(keys):
    b_io = lax.broadcasted_iota(jnp.int32, (B_ROWS, A_COLS), 0)
    a_io = lax.broadcasted_iota(jnp.int32, (B_ROWS, A_COLS), 1)
    idx = b_io * A_COLS + a_io
    k = 2
    while k <= A_COLS:
        j = k // 2
        while j >= 1:
            if k < A_COLS:
                desc = (a_io & k) == 0
            else:
                desc = (b_io & (B_ROWS // 2)) == 0
            keys, idx = _cmpx(keys, idx, j, a_io, desc)
            j //= 2
        k *= 2
    live = B_ROWS
    while live > 1:
        half = live // 2
        ak, ai = keys[0:half], idx[0:half]
        bk, bi = keys[half:live], idx[half:live]
        a_larger = (ak > bk) | ((ak == bk) & (ai < bi))
        keys = jnp.where(a_larger, ak, bk)
        idx = jnp.where(a_larger, ai, bi)
        a_io_h = lax.broadcasted_iota(jnp.int32, (half, A_COLS), 1)
        row_io_h = lax.broadcasted_iota(jnp.int32, (half, A_COLS), 0)
        desc = row_io_h < max(half // 2, 1)
        j = A_COLS // 2
        while j >= 1:
            keys, idx = _cmpx(keys, idx, j, a_io_h, desc)
            j //= 2
        live = half
    return keys, idx


def _char_sort_body(wf_ref, cf_ref, topv_ref, topi_ref):
    wf = wf_ref[...]
    cf = cf_ref[...]
    cscore = jnp.where((wf > WORD_MIN_SCORE) & (cf > CHAR_MIN_SCORE), cf, 0.0)
    keys, idx = _bitonic_topk(cscore)
    topv_ref[...] = keys[0:1, :]
    topi_ref[...] = idx[0:1, :]


def _word_sort_body(wf_ref, topv_ref, topi_ref):
    wf = wf_ref[...]
    wscore = jnp.where(wf > WORD_MIN_SCORE, wf, 0.0)
    keys, idx = _bitonic_topk(wscore)
    topv_ref[...] = keys[0:1, :]
    topi_ref[...] = idx[0:1, :]


def _sc_gather_body(cls_ref, cidx_ref, out_ref, cidx_v, idx_all, rows_v, sem):
    wid = lax.axis_index("s") * NC + lax.axis_index("c")
    base = wid * BPW
    pltpu.sync_copy(cidx_ref.at[pl.ds(base, BPW)], cidx_v)
    for ch in range(NUM_CHAR_CLASS):
        off = jnp.int32(ch * N)
        for g in range(BPW // L):
            idx_all[ch, pl.ds(g * L, L)] = cidx_v[pl.ds(g * L, L)] + off
    copies = [
        pltpu.async_copy(cls_ref.at[idx_all.at[ch]], rows_v.at[ch], sem)
        for ch in range(NUM_CHAR_CLASS)
    ]
    for cp in copies:
        cp.wait()
    pltpu.sync_copy(rows_v, out_ref.at[wid])


@functools.cache
def _sc_gather_kernel():
    return pl.kernel(
        _sc_gather_body,
        out_type=jax.ShapeDtypeStruct((NW, NUM_CHAR_CLASS, BPW), jnp.float32),
        mesh=plsc.VectorSubcoreMesh(core_axis_name="c", subcore_axis_name="s",
                                    num_cores=NC, num_subcores=NS),
        scratch_types=[
            pltpu.VMEM((BPW,), jnp.int32),
            pltpu.VMEM((NUM_CHAR_CLASS, BPW), jnp.int32),
            pltpu.VMEM((NUM_CHAR_CLASS, BPW), jnp.float32),
            pltpu.SemaphoreType.DMA,
        ],
    )


def _sc_gather(cls_flat, cidx):
    return _sc_gather_kernel()(cls_flat, cidx)


def _transpose(x):
    return x.T


def _gather_feats(stack, topi, n_ch):
    r = topi // WMAP
    c = topi % WMAP
    sub_io = lax.broadcasted_iota(jnp.int32, (HMAP, K), 0)
    row_sel = jnp.where(sub_io == jnp.broadcast_to(r, (HMAP, K)), 1.0, 0.0)
    col_sel = jnp.where(sub_io == jnp.broadcast_to(c, (HMAP, K)), 1.0, 0.0)
    h = jnp.dot(stack, col_sel, preferred_element_type=jnp.float32,
                precision=lax.Precision.HIGHEST)
    feats = []
    for ci in range(n_ch):
        hm = h[ci * HMAP:(ci + 1) * HMAP, :] * row_sel
        feats.append(jnp.sum(hm, axis=0, keepdims=True))
    return feats


def _nms_and_pack(qx, qy, topv, iou_thresh, wm1, hm1):
    bx1 = jnp.minimum(jnp.minimum(qx[0], qx[1]), jnp.minimum(qx[2], qx[3]))
    bx2 = jnp.maximum(jnp.maximum(qx[0], qx[1]), jnp.maximum(qx[2], qx[3]))
    by1 = jnp.minimum(jnp.minimum(qy[0], qy[1]), jnp.minimum(qy[2], qy[3]))
    by2 = jnp.maximum(jnp.maximum(qy[0], qy[1]), jnp.maximum(qy[2], qy[3]))
    area = (bx2 - bx1) * (by2 - by1)
    validf = topv > 0.0

    cols = jnp.concatenate([bx1, by1, bx2, by2, area, topv], axis=0)
    cols_t = _transpose(cols)
    bx1c = cols_t[:, 0:1]
    by1c = cols_t[:, 1:2]
    bx2c = cols_t[:, 2:3]
    by2c = cols_t[:, 3:4]
    areac = cols_t[:, 4:5]
    validc = cols_t[:, 5:6] > 0.0

    ix1 = jnp.maximum(bx1c, bx1)
    iy1 = jnp.maximum(by1c, by1)
    ix2 = jnp.minimum(bx2c, bx2)
    iy2 = jnp.minimum(by2c, by2)
    inter = jnp.maximum(ix2 - ix1, 0.0) * jnp.maximum(iy2 - iy1, 0.0)
    iou = inter / (areac + area - inter + 1e-6)

    sub_io = lax.broadcasted_iota(jnp.int32, (K, K), 0)
    lane_io = lax.broadcasted_iota(jnp.int32, (K, K), 1)
    sup = (iou > iou_thresh) & (sub_io < lane_io) & validc
    supf = jnp.where(sup, 1.0, 0.0)
    suppressed = jnp.max(supf, axis=0, keepdims=True)
    keepf = jnp.where(validf & (suppressed < 0.5), 1.0, 0.0)

    rows = []
    for i in range(4):
        rows.append(jnp.clip(jnp.round(qx[i]), 0.0, wm1))
        rows.append(jnp.clip(jnp.round(qy[i]), 0.0, hm1))
    rows.append(topv)
    out9 = jnp.concatenate(rows, axis=0) * keepf
    return out9, keepf


def _box_body(wtopv_ref, wtopi_ref, ctopv_ref, ctopi_ref,
              wstack_ref, cstack_ref, params_ref,
              cb_ref, wb_ref, keep_ref):
    params = params_ref[...]
    sw4 = params[:, 0:1] * STRIDE
    sh4 = params[:, 1:2] * STRIDE
    wm1 = params[:, 2:3]
    hm1 = params[:, 3:4]

    wtopv = wtopv_ref[...]
    wtopi = wtopi_ref[...]
    t, b, l, r, orient = _gather_feats(wstack_ref[...], wtopi, 5)
    xs = (wtopi % WMAP).astype(jnp.float32)
    ys = (wtopi // WMAP).astype(jnp.float32)
    x1 = sw4 * (xs - l)
    y1 = sh4 * (ys - t)
    x2 = sw4 * (xs + r)
    y2 = sh4 * (ys + b)
    cx = sw4 * xs
    cy = sh4 * ys
    cosv = jnp.cos(orient)
    sinv = jnp.sin(orient)
    dx1 = x1 - cx
    dx2 = x2 - cx
    dy1 = y1 - cy
    dy2 = y2 - cy
    qx = [cx + cosv * dx1 - sinv * dy1,
          cx + cosv * dx2 - sinv * dy1,
          cx + cosv * dx2 - sinv * dy2,
          cx + cosv * dx1 - sinv * dy2]
    qy = [cy + sinv * dx1 + cosv * dy1,
          cy + sinv * dx2 + cosv * dy1,
          cy + sinv * dx2 + cosv * dy2,
          cy + sinv * dx1 + cosv * dy2]
    wout9, _ = _nms_and_pack(qx, qy, wtopv, WORD_NMS_IOU, wm1, hm1)
    wb_ref[...] = _transpose(wout9)

    ctopv = ctopv_ref[...]
    ctopi = ctopi_ref[...]
    ct, cb, cl, cr = _gather_feats(cstack_ref[...], ctopi, 4)
    cxs = (ctopi % WMAP).astype(jnp.float32)
    cys = (ctopi // WMAP).astype(jnp.float32)
    cx1 = sw4 * (cxs - cl)
    cy1 = sh4 * (cys - ct)
    cx2 = sw4 * (cxs + cr)
    cy2 = sh4 * (cys + cb)
    cqx = [cx1, cx2, cx2, cx1]
    cqy = [cy1, cy1, cy2, cy2]
    cout9, ckeep = _nms_and_pack(cqx, cqy, ctopv, CHAR_NMS_IOU, wm1, hm1)
    cb_ref[...] = _transpose(cout9)
    keep_ref[...] = ckeep


def _finish_body(cls_ref, keep_ref, cs_ref):
    cs_ref[...] = _transpose(cls_ref[...] * keep_ref[...])


@functools.partial(jax.jit, static_argnums=())
def kernel(pred_word_fg, pred_word_tblr, pred_word_orient, pred_char_fg,
           pred_char_tblr, pred_char_cls, im_scale_w, im_scale_h,
           original_im_w, original_im_h):
    wf = pred_word_fg.reshape(B_ROWS, A_COLS)
    cf = pred_char_fg.reshape(B_ROWS, A_COLS)
    cls_flat = pred_char_cls.reshape(NUM_CHAR_CLASS * N)
    wstack = jnp.concatenate(
        [pred_word_tblr, pred_word_orient[None]], axis=0
    ).reshape(5 * HMAP, WMAP)
    cstack = pred_char_tblr.reshape(4 * HMAP, WMAP)
    wm1 = jnp.asarray(original_im_w, jnp.float32) - 1.0
    hm1 = jnp.asarray(original_im_h, jnp.float32) - 1.0
    params = jnp.stack([
        jnp.asarray(im_scale_w, jnp.float32),
        jnp.asarray(im_scale_h, jnp.float32),
        wm1, hm1,
    ]).reshape(1, 4)

    ctopv, ctopi = pl.pallas_call(
        _char_sort_body,
        out_shape=(
            jax.ShapeDtypeStruct((1, K), jnp.float32),
            jax.ShapeDtypeStruct((1, K), jnp.int32),
        ),
    )(wf, cf)

    cls3 = _sc_gather(cls_flat, ctopi.reshape(K))

    wtopv, wtopi = pl.pallas_call(
        _word_sort_body,
        out_shape=(
            jax.ShapeDtypeStruct((1, K), jnp.float32),
            jax.ShapeDtypeStruct((1, K), jnp.int32),
        ),
    )(wf)

    char_bboxes, word_bboxes, ckeep = pl.pallas_call(
        _box_body,
        out_shape=(
            jax.ShapeDtypeStruct((K, 9), jnp.float32),
            jax.ShapeDtypeStruct((K, 9), jnp.float32),
            jax.ShapeDtypeStruct((1, K), jnp.float32),
        ),
    )(wtopv, wtopi, ctopv, ctopi, wstack, cstack, params)

    cls2d = cls3.transpose(1, 0, 2).reshape(NUM_CHAR_CLASS, K)
    char_scores = pl.pallas_call(
        _finish_body,
        out_shape=jax.ShapeDtypeStruct((K, NUM_CHAR_CLASS), jnp.float32),
    )(cls2d, ckeep)
    return (char_bboxes, char_scores, word_bboxes)


# speedup vs baseline: 1.1870x; 1.0469x over previous
import functools

import jax
import jax.numpy as jnp
from jax import lax
from jax.experimental import pallas as pl
from jax.experimental.pallas import tpu as pltpu
from jax.experimental.pallas import tpu_sc as plsc

HMAP = 128
WMAP = 128
N = HMAP * WMAP
K = 1024
B_ROWS = 16
A_COLS = 1024

WORD_MIN_SCORE = 0.5
CHAR_MIN_SCORE = 0.25
WORD_NMS_IOU = 0.5
CHAR_NMS_IOU = 0.3
NUM_CHAR_CLASS = 68
STRIDE = 4.0

NC, NS, L = 2, 16, 16
NW = NC * NS
BPW = K // NW


def _roll(x, s, axis):
    n = x.shape[axis]
    s = s % n
    if s == 0:
        return x
    lo = [slice(None)] * x.ndim
    hi = [slice(None)] * x.ndim
    lo[axis] = slice(s, n)
    hi[axis] = slice(0, s)
    return jnp.concatenate([x[tuple(lo)], x[tuple(hi)]], axis=axis)


def _cmpx(keys, idx, j, a_io, desc):
    has_bit = (a_io & j) != 0
    pk = jnp.where(has_bit, _roll(keys, -j, 1), _roll(keys, j, 1))
    pi = jnp.where(has_bit, _roll(idx, -j, 1), _roll(idx, j, 1))
    mine_larger = (keys > pk) | ((keys == pk) & (idx < pi))
    take_mine = mine_larger == (desc != has_bit)
    return jnp.where(take_mine, keys, pk), jnp.where(take_mine, idx, pi)


def _bitonic_topk(keys):
    b_io = lax.broadcasted_iota(jnp.int32, (B_ROWS, A_COLS), 0)
    a_io = lax.broadcasted_iota(jnp.int32, (B_ROWS, A_COLS), 1)
    idx = b_io * A_COLS + a_io
    k = 2
    while k <= A_COLS:
        j = k // 2
        while j >= 1:
            if k < A_COLS:
                desc = (a_io & k) == 0
            else:
                desc = (b_io & (B_ROWS // 2)) == 0
            keys, idx = _cmpx(keys, idx, j, a_io, desc)
            j //= 2
        k *= 2
    live = B_ROWS
    while live > 1:
        half = live // 2
        ak, ai = keys[0:half], idx[0:half]
        bk, bi = keys[half:live], idx[half:live]
        a_larger = (ak > bk) | ((ak == bk) & (ai < bi))
        keys = jnp.where(a_larger, ak, bk)
        idx = jnp.where(a_larger, ai, bi)
        a_io_h = lax.broadcasted_iota(jnp.int32, (half, A_COLS), 1)
        row_io_h = lax.broadcasted_iota(jnp.int32, (half, A_COLS), 0)
        desc = row_io_h < max(half // 2, 1)
        j = A_COLS // 2
        while j >= 1:
            keys, idx = _cmpx(keys, idx, j, a_io_h, desc)
            j //= 2
        live = half
    return keys, idx


def _char_sort_body(wf_ref, cf_ref, topv_ref, topi_ref):
    wf = wf_ref[...]
    cf = cf_ref[...]
    cscore = jnp.where((wf > WORD_MIN_SCORE) & (cf > CHAR_MIN_SCORE), cf, 0.0)
    keys, idx = _bitonic_topk(cscore)
    topv_ref[...] = keys[0:1, :]
    topi_ref[...] = idx[0:1, :]


def _sc_gather_body(cls_ref, cidx_ref, out_ref, cidx_v, idx_all, rows_v, sem):
    wid = lax.axis_index("s") * NC + lax.axis_index("c")
    base = wid * BPW
    pltpu.sync_copy(cidx_ref.at[pl.ds(base, BPW)], cidx_v)
    for ch in range(NUM_CHAR_CLASS):
        off = jnp.int32(ch * N)
        for g in range(BPW // L):
            idx_all[ch, pl.ds(g * L, L)] = cidx_v[pl.ds(g * L, L)] + off
    copies = [
        pltpu.async_copy(cls_ref.at[idx_all.at[ch]], rows_v.at[ch], sem)
        for ch in range(NUM_CHAR_CLASS)
    ]
    for cp in copies:
        cp.wait()
    pltpu.sync_copy(rows_v, out_ref.at[wid])


@functools.cache
def _sc_gather_kernel():
    return pl.kernel(
        _sc_gather_body,
        out_type=jax.ShapeDtypeStruct((NW, NUM_CHAR_CLASS, BPW), jnp.float32),
        mesh=plsc.VectorSubcoreMesh(core_axis_name="c", subcore_axis_name="s",
                                    num_cores=NC, num_subcores=NS),
        scratch_types=[
            pltpu.VMEM((BPW,), jnp.int32),
            pltpu.VMEM((NUM_CHAR_CLASS, BPW), jnp.int32),
            pltpu.VMEM((NUM_CHAR_CLASS, BPW), jnp.float32),
            pltpu.SemaphoreType.DMA,
        ],
    )


def _sc_gather(cls_flat, cidx):
    return _sc_gather_kernel()(cls_flat, cidx)


def _transpose(x):
    return x.T


def _gather_feats(stack, topi, n_ch):
    r = topi // WMAP
    c = topi % WMAP
    sub_io = lax.broadcasted_iota(jnp.int32, (HMAP, K), 0)
    row_sel = jnp.where(sub_io == jnp.broadcast_to(r, (HMAP, K)), 1.0, 0.0)
    col_sel = jnp.where(sub_io == jnp.broadcast_to(c, (HMAP, K)), 1.0, 0.0)
    h = jnp.dot(stack, col_sel, preferred_element_type=jnp.float32,
                precision=lax.Precision.HIGHEST)
    feats = []
    for ci in range(n_ch):
        hm = h[ci * HMAP:(ci + 1) * HMAP, :] * row_sel
        feats.append(jnp.sum(hm, axis=0, keepdims=True))
    return feats


def _nms_and_pack(qx, qy, topv, iou_thresh, wm1, hm1):
    bx1 = jnp.minimum(jnp.minimum(qx[0], qx[1]), jnp.minimum(qx[2], qx[3]))
    bx2 = jnp.maximum(jnp.maximum(qx[0], qx[1]), jnp.maximum(qx[2], qx[3]))
    by1 = jnp.minimum(jnp.minimum(qy[0], qy[1]), jnp.minimum(qy[2], qy[3]))
    by2 = jnp.maximum(jnp.maximum(qy[0], qy[1]), jnp.maximum(qy[2], qy[3]))
    area = (bx2 - bx1) * (by2 - by1)
    validf = topv > 0.0

    cols = jnp.concatenate([bx1, by1, bx2, by2, area, topv], axis=0)
    cols_t = _transpose(cols)
    bx1c = cols_t[:, 0:1]
    by1c = cols_t[:, 1:2]
    bx2c = cols_t[:, 2:3]
    by2c = cols_t[:, 3:4]
    areac = cols_t[:, 4:5]
    validc = cols_t[:, 5:6] > 0.0

    ix1 = jnp.maximum(bx1c, bx1)
    iy1 = jnp.maximum(by1c, by1)
    ix2 = jnp.minimum(bx2c, bx2)
    iy2 = jnp.minimum(by2c, by2)
    inter = jnp.maximum(ix2 - ix1, 0.0) * jnp.maximum(iy2 - iy1, 0.0)
    iou = inter / (areac + area - inter + 1e-6)

    sub_io = lax.broadcasted_iota(jnp.int32, (K, K), 0)
    lane_io = lax.broadcasted_iota(jnp.int32, (K, K), 1)
    sup = (iou > iou_thresh) & (sub_io < lane_io) & validc
    supf = jnp.where(sup, 1.0, 0.0)
    suppressed = jnp.max(supf, axis=0, keepdims=True)
    keepf = jnp.where(validf & (suppressed < 0.5), 1.0, 0.0)

    rows = []
    for i in range(4):
        rows.append(jnp.clip(jnp.round(qx[i]), 0.0, wm1))
        rows.append(jnp.clip(jnp.round(qy[i]), 0.0, hm1))
    rows.append(topv)
    out9 = jnp.concatenate(rows, axis=0) * keepf
    return out9, keepf


def _box_body(wf_ref, ctopv_ref, ctopi_ref,
              wstack_ref, cstack_ref, params_ref,
              cb_ref, wb_ref, keep_ref):
    params = params_ref[...]
    sw4 = params[:, 0:1] * STRIDE
    sh4 = params[:, 1:2] * STRIDE
    wm1 = params[:, 2:3]
    hm1 = params[:, 3:4]

    wf = wf_ref[...]
    wscore = jnp.where(wf > WORD_MIN_SCORE, wf, 0.0)
    wkeys, widx = _bitonic_topk(wscore)
    wtopv = wkeys[0:1, :]
    wtopi = widx[0:1, :]
    t, b, l, r, orient = _gather_feats(wstack_ref[...], wtopi, 5)
    xs = (wtopi % WMAP).astype(jnp.float32)
    ys = (wtopi // WMAP).astype(jnp.float32)
    x1 = sw4 * (xs - l)
    y1 = sh4 * (ys - t)
    x2 = sw4 * (xs + r)
    y2 = sh4 * (ys + b)
    cx = sw4 * xs
    cy = sh4 * ys
    cosv = jnp.cos(orient)
    sinv = jnp.sin(orient)
    dx1 = x1 - cx
    dx2 = x2 - cx
    dy1 = y1 - cy
    dy2 = y2 - cy
    qx = [cx + cosv * dx1 - sinv * dy1,
          cx + cosv * dx2 - sinv * dy1,
          cx + cosv * dx2 - sinv * dy2,
          cx + cosv * dx1 - sinv * dy2]
    qy = [cy + sinv * dx1 + cosv * dy1,
          cy + sinv * dx2 + cosv * dy1,
          cy + sinv * dx2 + cosv * dy2,
          cy + sinv * dx1 + cosv * dy2]
    wout9, _ = _nms_and_pack(qx, qy, wtopv, WORD_NMS_IOU, wm1, hm1)
    wb_ref[...] = _transpose(wout9)

    ctopv = ctopv_ref[...]
    ctopi = ctopi_ref[...]
    ct, cb, cl, cr = _gather_feats(cstack_ref[...], ctopi, 4)
    cxs = (ctopi % WMAP).astype(jnp.float32)
    cys = (ctopi // WMAP).astype(jnp.float32)
    cx1 = sw4 * (cxs - cl)
    cy1 = sh4 * (cys - ct)
    cx2 = sw4 * (cxs + cr)
    cy2 = sh4 * (cys + cb)
    cqx = [cx1, cx2, cx2, cx1]
    cqy = [cy1, cy1, cy2, cy2]
    cout9, ckeep = _nms_and_pack(cqx, cqy, ctopv, CHAR_NMS_IOU, wm1, hm1)
    cb_ref[...] = _transpose(cout9)
    keep_ref[...] = ckeep


def _finish_body(cls_ref, keep_ref, cs_ref):
    cs_ref[...] = _transpose(cls_ref[...] * keep_ref[...])


@functools.partial(jax.jit, static_argnums=())
def kernel(pred_word_fg, pred_word_tblr, pred_word_orient, pred_char_fg,
           pred_char_tblr, pred_char_cls, im_scale_w, im_scale_h,
           original_im_w, original_im_h):
    wf = pred_word_fg.reshape(B_ROWS, A_COLS)
    cf = pred_char_fg.reshape(B_ROWS, A_COLS)
    cls_flat = pred_char_cls.reshape(NUM_CHAR_CLASS * N)
    wstack = jnp.concatenate(
        [pred_word_tblr, pred_word_orient[None]], axis=0
    ).reshape(5 * HMAP, WMAP)
    cstack = pred_char_tblr.reshape(4 * HMAP, WMAP)
    wm1 = jnp.asarray(original_im_w, jnp.float32) - 1.0
    hm1 = jnp.asarray(original_im_h, jnp.float32) - 1.0
    params = jnp.stack([
        jnp.asarray(im_scale_w, jnp.float32),
        jnp.asarray(im_scale_h, jnp.float32),
        wm1, hm1,
    ]).reshape(1, 4)

    ctopv, ctopi = pl.pallas_call(
        _char_sort_body,
        out_shape=(
            jax.ShapeDtypeStruct((1, K), jnp.float32),
            jax.ShapeDtypeStruct((1, K), jnp.int32),
        ),
    )(wf, cf)

    cls3 = _sc_gather(cls_flat, ctopi.reshape(K))

    char_bboxes, word_bboxes, ckeep = pl.pallas_call(
        _box_body,
        out_shape=(
            jax.ShapeDtypeStruct((K, 9), jnp.float32),
            jax.ShapeDtypeStruct((K, 9), jnp.float32),
            jax.ShapeDtypeStruct((1, K), jnp.float32),
        ),
    )(wf, ctopv, ctopi, wstack, cstack, params)

    cls2d = cls3.transpose(1, 0, 2).reshape(NUM_CHAR_CLASS, K)
    char_scores = pl.pallas_call(
        _finish_body,
        out_shape=jax.ShapeDtypeStruct((K, NUM_CHAR_CLASS), jnp.float32),
    )(cls2d, ckeep)
    return (char_bboxes, char_scores, word_bboxes)
